# baseline (speedup 1.0000x reference)
import functools

import jax
import jax.numpy as jnp
from jax import lax
from jax.experimental import pallas as pl
from jax.experimental.pallas import tpu as pltpu
from jax.experimental.pallas import tpu_sc as plsc

_E = 16
_D = 768
_F = 4 * _D
_N = 4096
_K = 2
_NP = _N * _K
_TBLK = 256
_FB = 128
_NB = _NP // _FB + _E
_NS = _NB * _FB


def _route_kernel(x_ref, wr_ref, dst_ref, gatev_ref, counts_ref, o_scr):
    x = x_ref[...]
    logits = lax.dot_general(x, wr_ref[...], (((1,), (1,)), ((), ())),
                             preferred_element_type=jnp.float32)
    m = jnp.max(logits, axis=-1, keepdims=True)
    p = jnp.exp(logits - m)
    p = p / jnp.sum(p, axis=-1, keepdims=True)
    eidx = lax.broadcasted_iota(jnp.int32, p.shape, 1)
    i1 = jnp.argmax(p, axis=-1)
    m1 = eidx == i1[:, None]
    p2 = jnp.where(m1, -jnp.inf, p)
    i2 = jnp.argmax(p2, axis=-1)
    m2 = eidx == i2[:, None]
    g1 = jnp.max(p, axis=-1, keepdims=True)
    g2 = jnp.max(p2, axis=-1, keepdims=True)
    gatev_ref[...] = jnp.concatenate([g1, g2], axis=1)

    m1f = m1.astype(jnp.float32)
    m2f = m2.astype(jnp.float32)
    o = jnp.concatenate([m1f[:, None, :], m2f[:, None, :]], axis=1)
    o = o.reshape(_NP, _E)
    o_scr[...] = o

    counts = jnp.sum(o, axis=0, keepdims=True)
    counts_ref[...] = counts.astype(jnp.int32)
    bpe = jnp.ceil(counts * (1.0 / _FB))
    tri_e = (lax.broadcasted_iota(jnp.int32, (_E, _E), 0)
             < lax.broadcasted_iota(jnp.int32, (_E, _E), 1)).astype(jnp.float32)
    start = float(_FB) * lax.dot_general(
        bpe, tri_e, (((1,), (0,)), ((), ())),
        preferred_element_type=jnp.float32)

    tri = (lax.broadcasted_iota(jnp.int32, (_TBLK, _TBLK), 1)
           < lax.broadcasted_iota(jnp.int32, (_TBLK, _TBLK), 0)).astype(jnp.float32)

    def body(c, carry):
        oc = o_scr[pl.ds(c * _TBLK, _TBLK), :]
        rc = carry + lax.dot_general(tri, oc, (((1,), (0,)), ((), ())),
                                     preferred_element_type=jnp.float32)
        dstc = jnp.sum(oc * (rc + start), axis=1, keepdims=True)
        dst_ref[pl.ds(c * _TBLK, _TBLK), :] = dstc.astype(jnp.int32)
        return carry + jnp.sum(oc, axis=0, keepdims=True)

    lax.fori_loop(0, _NP // _TBLK, body, jnp.zeros((1, _E), jnp.float32))


def _route(xf, W_router):
    return pl.pallas_call(
        _route_kernel,
        grid=(1,),
        in_specs=[
            pl.BlockSpec((_N, _D), lambda i: (0, 0)),
            pl.BlockSpec((_E, _D), lambda i: (0, 0)),
        ],
        out_specs=[
            pl.BlockSpec((_NP, 1), lambda i: (0, 0)),
            pl.BlockSpec((_N, _K), lambda i: (0, 0)),
            pl.BlockSpec((1, _E), lambda i: (0, 0)),
        ],
        out_shape=[
            jax.ShapeDtypeStruct((_NP, 1), jnp.int32),
            jax.ShapeDtypeStruct((_N, _K), jnp.float32),
            jax.ShapeDtypeStruct((1, _E), jnp.int32),
        ],
        scratch_shapes=[pltpu.VMEM((_NP, _E), jnp.float32)],
    )(xf, W_router)


_SC_CH = 64
_PW = 128
_R = _D // _PW


def _sc_dispatch(x6, dst6, tok6):
    info = plsc.get_sparse_core_info()
    nw = info.num_cores * info.num_subcores
    pairs_per_w = _NP // nw
    nch = pairs_per_w // _SC_CH
    cp = _SC_CH * _R
    mesh = plsc.VectorSubcoreMesh(core_axis_name="c", subcore_axis_name="s")

    @functools.partial(
        pl.kernel, mesh=mesh,
        out_type=jax.ShapeDtypeStruct((_NS * _R, _PW), jnp.float32),
        scratch_types=[
            pltpu.VMEM((cp,), jnp.int32),
            pltpu.VMEM((cp,), jnp.int32),
            pltpu.VMEM((cp, _PW), jnp.float32),
            pltpu.SemaphoreType.DMA,
        ],
    )
    def k(x_hbm, dst_hbm, tok_hbm, xs_hbm, tok_v, dst_v, rows_v, sem):
        wid = lax.axis_index("s") * info.num_cores + lax.axis_index("c")
        base = wid * pairs_per_w * _R

        for c in range(nch):
            cbase = base + c * cp
            pltpu.sync_copy(tok_hbm.at[pl.ds(cbase, cp)], tok_v)
            pltpu.sync_copy(dst_hbm.at[pl.ds(cbase, cp)], dst_v)
            pltpu.async_copy(x_hbm.at[tok_v], rows_v, sem).wait()
            pltpu.async_copy(rows_v, xs_hbm.at[dst_v], sem).wait()

    return k(x6, dst6, tok6)


def _ffn_kernel(be_ref, xs_ref, w1_ref, b1_ref, w2_ref, b2_ref, po_ref):
    xb = xs_ref[...]
    h = lax.dot_general(xb, w1_ref[0], (((1,), (1,)), ((), ())),
                        preferred_element_type=jnp.float32) + b1_ref[0]
    h = 0.5 * h * (1.0 + lax.erf(h * 0.7071067811865476))
    po_ref[...] = lax.dot_general(h, w2_ref[0], (((1,), (1,)), ((), ())),
                                  preferred_element_type=jnp.float32) + b2_ref[0]


def _ffn(block_expert, xs, W1, b1, W2, b2):
    grid_spec = pltpu.PrefetchScalarGridSpec(
        num_scalar_prefetch=1,
        grid=(_NB,),
        in_specs=[
            pl.BlockSpec((_FB, _D), lambda b, be: (b, 0)),
            pl.BlockSpec((1, _F, _D), lambda b, be: (be[b], 0, 0)),
            pl.BlockSpec((1, 1, _F), lambda b, be: (be[b], 0, 0)),
            pl.BlockSpec((1, _D, _F), lambda b, be: (be[b], 0, 0)),
            pl.BlockSpec((1, 1, _D), lambda b, be: (be[b], 0, 0)),
        ],
        out_specs=pl.BlockSpec((_FB, _D), lambda b, be: (b, 0)),
    )
    return pl.pallas_call(
        _ffn_kernel,
        grid_spec=grid_spec,
        out_shape=jax.ShapeDtypeStruct((_NS, _D), jnp.float32),
    )(block_expert, xs, W1, b1.reshape(_E, 1, _F), W2, b2.reshape(_E, 1, _D))


_CB_T = 16


def _sc_combine(x6, po6, dst6, gates16):
    info = plsc.get_sparse_core_info()
    nw = info.num_cores * info.num_subcores
    tok_per_w = _N // nw
    nch = tok_per_w // _CB_T
    tp = _CB_T * _R
    pp = 2 * _CB_T * _R
    mesh = plsc.VectorSubcoreMesh(core_axis_name="c", subcore_axis_name="s")

    @functools.partial(
        pl.kernel, mesh=mesh,
        out_type=jax.ShapeDtypeStruct((_N * _R, _PW), jnp.float32),
        scratch_types=[
            pltpu.VMEM((pp,), jnp.int32),
            pltpu.VMEM((2 * _CB_T, 16), jnp.float32),
            pltpu.VMEM((pp, _PW), jnp.float32),
            pltpu.VMEM((tp, _PW), jnp.float32),
            pltpu.SemaphoreType.DMA,
        ],
    )
    def k(x_hbm, po_hbm, dst_hbm, g_hbm, y_hbm, dst_v, g_v, po_v, acc_v, sem):
        wid = lax.axis_index("s") * info.num_cores + lax.axis_index("c")
        base_t = wid * tok_per_w

        def chunk(c, _):
            tb = base_t + c * _CB_T
            pltpu.sync_copy(dst_hbm.at[pl.ds(2 * tb * _R, pp)], dst_v)
            pltpu.sync_copy(g_hbm.at[pl.ds(2 * tb, 2 * _CB_T), :], g_v)
            pltpu.async_copy(po_hbm.at[dst_v], po_v, sem).wait()
            pltpu.sync_copy(x_hbm.at[pl.ds(tb * _R, tp)], acc_v)

            def tok(j, _2):
                ga = g_v[2 * j, :]
                gb = g_v[2 * j + 1, :]
                for r in range(_R):
                    for v in range(8):
                        sl = pl.ds(v * 16, 16)
                        acc_v[j * _R + r, sl] = (
                            acc_v[j * _R + r, sl]
                            + ga * po_v[2 * j * _R + r, sl]
                            + gb * po_v[(2 * j + 1) * _R + r, sl])
                return 0

            lax.fori_loop(0, _CB_T, tok, 0)
            pltpu.sync_copy(acc_v, y_hbm.at[pl.ds(tb * _R, tp)])
            return 0

        lax.fori_loop(0, nch, chunk, 0)

    return k(x6, po6, dst6, gates16)


def kernel(x, W_router, W1, b1, W2, b2):
    B, L, D = x.shape
    xf = x.reshape(_N, D)

    dst2d, gatev, counts = _route(xf, W_router)
    dst = dst2d.reshape(_NP)
    gates = gatev.reshape(_NP)

    cnt = counts.reshape(_E)
    cb = jnp.cumsum((cnt + _FB - 1) // _FB)
    block_expert = jnp.clip(
        jnp.searchsorted(cb, jnp.arange(_NB, dtype=jnp.int32), side="right"),
        0, _E - 1).astype(jnp.int32)

    piece = jnp.arange(_R, dtype=jnp.int32)
    tok6 = ((jnp.arange(_NP, dtype=jnp.int32) // _K * _R)[:, None]
            + piece).reshape(-1)
    dst6 = ((dst * _R)[:, None] + piece).reshape(-1)
    gates16 = jnp.broadcast_to(gates[:, None], (_NP, 16))

    x6 = xf.reshape(_N * _R, _PW)
    xs = _sc_dispatch(x6, dst6, tok6)
    po = _ffn(block_expert, xs.reshape(_NS, _D), W1, b1, W2, b2)
    y = _sc_combine(x6, po.reshape(_NS * _R, _PW), dst6, gates16)
    return y.reshape(B, L, D)


# speedup vs baseline: 1.0893x; 1.0893x over previous
import functools

import jax
import jax.numpy as jnp
from jax import lax
from jax.experimental import pallas as pl
from jax.experimental.pallas import tpu as pltpu
from jax.experimental.pallas import tpu_sc as plsc

_E = 16
_D = 768
_F = 4 * _D
_N = 4096
_K = 2
_NP = _N * _K
_TBLK = 256
_FB = 256
_NB = _NP // _FB + _E
_NS = _NB * _FB


def _route_kernel(x_ref, wr_ref, dst_ref, gatev_ref, counts_ref, o_scr):
    x = x_ref[...]
    logits = lax.dot_general(x, wr_ref[...], (((1,), (1,)), ((), ())),
                             preferred_element_type=jnp.float32)
    m = jnp.max(logits, axis=-1, keepdims=True)
    p = jnp.exp(logits - m)
    p = p / jnp.sum(p, axis=-1, keepdims=True)
    eidx = lax.broadcasted_iota(jnp.int32, p.shape, 1)
    i1 = jnp.argmax(p, axis=-1)
    m1 = eidx == i1[:, None]
    p2 = jnp.where(m1, -jnp.inf, p)
    i2 = jnp.argmax(p2, axis=-1)
    m2 = eidx == i2[:, None]
    g1 = jnp.max(p, axis=-1, keepdims=True)
    g2 = jnp.max(p2, axis=-1, keepdims=True)
    gatev_ref[...] = jnp.concatenate([g1, g2], axis=1)

    m1f = m1.astype(jnp.float32)
    m2f = m2.astype(jnp.float32)
    o = jnp.concatenate([m1f[:, None, :], m2f[:, None, :]], axis=1)
    o = o.reshape(_NP, _E)
    o_scr[...] = o

    counts = jnp.sum(o, axis=0, keepdims=True)
    counts_ref[...] = counts.astype(jnp.int32)
    bpe = jnp.ceil(counts * (1.0 / _FB))
    tri_e = (lax.broadcasted_iota(jnp.int32, (_E, _E), 0)
             < lax.broadcasted_iota(jnp.int32, (_E, _E), 1)).astype(jnp.float32)
    start = float(_FB) * lax.dot_general(
        bpe, tri_e, (((1,), (0,)), ((), ())),
        preferred_element_type=jnp.float32)

    tri = (lax.broadcasted_iota(jnp.int32, (_TBLK, _TBLK), 1)
           < lax.broadcasted_iota(jnp.int32, (_TBLK, _TBLK), 0)).astype(jnp.float32)

    def body(c, carry):
        oc = o_scr[pl.ds(c * _TBLK, _TBLK), :]
        rc = carry + lax.dot_general(tri, oc, (((1,), (0,)), ((), ())),
                                     preferred_element_type=jnp.float32)
        dstc = jnp.sum(oc * (rc + start), axis=1, keepdims=True)
        dst_ref[pl.ds(c * _TBLK, _TBLK), :] = dstc.astype(jnp.int32)
        return carry + jnp.sum(oc, axis=0, keepdims=True)

    lax.fori_loop(0, _NP // _TBLK, body, jnp.zeros((1, _E), jnp.float32))


def _route(xf, W_router):
    return pl.pallas_call(
        _route_kernel,
        grid=(1,),
        in_specs=[
            pl.BlockSpec((_N, _D), lambda i: (0, 0)),
            pl.BlockSpec((_E, _D), lambda i: (0, 0)),
        ],
        out_specs=[
            pl.BlockSpec((_NP, 1), lambda i: (0, 0)),
            pl.BlockSpec((_N, _K), lambda i: (0, 0)),
            pl.BlockSpec((1, _E), lambda i: (0, 0)),
        ],
        out_shape=[
            jax.ShapeDtypeStruct((_NP, 1), jnp.int32),
            jax.ShapeDtypeStruct((_N, _K), jnp.float32),
            jax.ShapeDtypeStruct((1, _E), jnp.int32),
        ],
        scratch_shapes=[pltpu.VMEM((_NP, _E), jnp.float32)],
    )(xf, W_router)


_SC_CH = 64
_PW = 128
_R = _D // _PW


def _sc_dispatch(x6, dst6, tok6):
    info = plsc.get_sparse_core_info()
    nw = info.num_cores * info.num_subcores
    pairs_per_w = _NP // nw
    nch = pairs_per_w // _SC_CH
    cp = _SC_CH * _R
    mesh = plsc.VectorSubcoreMesh(core_axis_name="c", subcore_axis_name="s")

    @functools.partial(
        pl.kernel, mesh=mesh,
        out_type=jax.ShapeDtypeStruct((_NS * _R, _PW), jnp.float32),
        scratch_types=[
            pltpu.VMEM((cp,), jnp.int32),
            pltpu.VMEM((cp,), jnp.int32),
            pltpu.VMEM((cp, _PW), jnp.float32),
            pltpu.SemaphoreType.DMA,
        ],
    )
    def k(x_hbm, dst_hbm, tok_hbm, xs_hbm, tok_v, dst_v, rows_v, sem):
        wid = lax.axis_index("s") * info.num_cores + lax.axis_index("c")
        base = wid * pairs_per_w * _R

        for c in range(nch):
            cbase = base + c * cp
            pltpu.sync_copy(tok_hbm.at[pl.ds(cbase, cp)], tok_v)
            pltpu.sync_copy(dst_hbm.at[pl.ds(cbase, cp)], dst_v)
            pltpu.async_copy(x_hbm.at[tok_v], rows_v, sem).wait()
            pltpu.async_copy(rows_v, xs_hbm.at[dst_v], sem).wait()

    return k(x6, dst6, tok6)


def _ffn_kernel(be_ref, xs_ref, w1_ref, b1_ref, w2_ref, b2_ref, po_ref):
    xb = xs_ref[...].astype(jnp.bfloat16)
    h = lax.dot_general(xb, w1_ref[0], (((1,), (1,)), ((), ())),
                        preferred_element_type=jnp.float32) + b1_ref[0]
    h = 0.5 * h * (1.0 + lax.erf(h * 0.7071067811865476))
    po_ref[...] = lax.dot_general(h.astype(jnp.bfloat16), w2_ref[0],
                                  (((1,), (1,)), ((), ())),
                                  preferred_element_type=jnp.float32) + b2_ref[0]


def _ffn(block_expert, xs, W1, b1, W2, b2):
    grid_spec = pltpu.PrefetchScalarGridSpec(
        num_scalar_prefetch=1,
        grid=(_NB,),
        in_specs=[
            pl.BlockSpec((_FB, _D), lambda b, be: (b, 0)),
            pl.BlockSpec((1, _F, _D), lambda b, be: (be[b], 0, 0)),
            pl.BlockSpec((1, 1, _F), lambda b, be: (be[b], 0, 0)),
            pl.BlockSpec((1, _D, _F), lambda b, be: (be[b], 0, 0)),
            pl.BlockSpec((1, 1, _D), lambda b, be: (be[b], 0, 0)),
        ],
        out_specs=pl.BlockSpec((_FB, _D), lambda b, be: (b, 0)),
    )
    return pl.pallas_call(
        _ffn_kernel,
        grid_spec=grid_spec,
        out_shape=jax.ShapeDtypeStruct((_NS, _D), jnp.float32),
    )(block_expert, xs, W1.astype(jnp.bfloat16), b1.reshape(_E, 1, _F),
      W2.astype(jnp.bfloat16), b2.reshape(_E, 1, _D))


_CB_T = 16


def _sc_combine(x6, po6, dst6, gates16):
    info = plsc.get_sparse_core_info()
    nw = info.num_cores * info.num_subcores
    tok_per_w = _N // nw
    nch = tok_per_w // _CB_T
    tp = _CB_T * _R
    pp = 2 * _CB_T * _R
    mesh = plsc.VectorSubcoreMesh(core_axis_name="c", subcore_axis_name="s")

    @functools.partial(
        pl.kernel, mesh=mesh,
        out_type=jax.ShapeDtypeStruct((_N * _R, _PW), jnp.float32),
        scratch_types=[
            pltpu.VMEM((pp,), jnp.int32),
            pltpu.VMEM((2 * _CB_T, 16), jnp.float32),
            pltpu.VMEM((pp, _PW), jnp.float32),
            pltpu.VMEM((tp, _PW), jnp.float32),
            pltpu.SemaphoreType.DMA,
        ],
    )
    def k(x_hbm, po_hbm, dst_hbm, g_hbm, y_hbm, dst_v, g_v, po_v, acc_v, sem):
        wid = lax.axis_index("s") * info.num_cores + lax.axis_index("c")
        base_t = wid * tok_per_w

        def chunk(c, _):
            tb = base_t + c * _CB_T
            pltpu.sync_copy(dst_hbm.at[pl.ds(2 * tb * _R, pp)], dst_v)
            pltpu.sync_copy(g_hbm.at[pl.ds(2 * tb, 2 * _CB_T), :], g_v)
            pltpu.async_copy(po_hbm.at[dst_v], po_v, sem).wait()
            pltpu.sync_copy(x_hbm.at[pl.ds(tb * _R, tp)], acc_v)

            def tok(j, _2):
                ga = g_v[2 * j, :]
                gb = g_v[2 * j + 1, :]
                for r in range(_R):
                    for v in range(8):
                        sl = pl.ds(v * 16, 16)
                        acc_v[j * _R + r, sl] = (
                            acc_v[j * _R + r, sl]
                            + ga * po_v[2 * j * _R + r, sl]
                            + gb * po_v[(2 * j + 1) * _R + r, sl])
                return 0

            lax.fori_loop(0, _CB_T, tok, 0)
            pltpu.sync_copy(acc_v, y_hbm.at[pl.ds(tb * _R, tp)])
            return 0

        lax.fori_loop(0, nch, chunk, 0)

    return k(x6, po6, dst6, gates16)


def kernel(x, W_router, W1, b1, W2, b2):
    B, L, D = x.shape
    xf = x.reshape(_N, D)

    dst2d, gatev, counts = _route(xf, W_router)
    dst = dst2d.reshape(_NP)
    gates = gatev.reshape(_NP)

    cnt = counts.reshape(_E)
    cb = jnp.cumsum((cnt + _FB - 1) // _FB)
    block_expert = jnp.clip(
        jnp.searchsorted(cb, jnp.arange(_NB, dtype=jnp.int32), side="right"),
        0, _E - 1).astype(jnp.int32)

    piece = jnp.arange(_R, dtype=jnp.int32)
    tok6 = ((jnp.arange(_NP, dtype=jnp.int32) // _K * _R)[:, None]
            + piece).reshape(-1)
    dst6 = ((dst * _R)[:, None] + piece).reshape(-1)
    gates16 = jnp.broadcast_to(gates[:, None], (_NP, 16))

    x6 = xf.reshape(_N * _R, _PW)
    xs = _sc_dispatch(x6, dst6, tok6)
    po = _ffn(block_expert, xs.reshape(_NS, _D), W1, b1, W2, b2)
    y = _sc_combine(x6, po.reshape(_NS * _R, _PW), dst6, gates16)
    return y.reshape(B, L, D)


# speedup vs baseline: 1.2856x; 1.1802x over previous
import functools

import jax
import jax.numpy as jnp
from jax import lax
from jax.experimental import pallas as pl
from jax.experimental.pallas import tpu as pltpu
from jax.experimental.pallas import tpu_sc as plsc

_E = 16
_D = 768
_F = 4 * _D
_N = 4096
_K = 2
_NP = _N * _K
_TBLK = 256
_FB = 512
_NB = _NP // _FB + _E
_NS = _NB * _FB


def _route_kernel(x_ref, wr_ref, dst_ref, gatev_ref, counts_ref, o_scr):
    x = x_ref[...]
    logits = lax.dot_general(x, wr_ref[...], (((1,), (1,)), ((), ())),
                             preferred_element_type=jnp.float32)
    m = jnp.max(logits, axis=-1, keepdims=True)
    p = jnp.exp(logits - m)
    p = p / jnp.sum(p, axis=-1, keepdims=True)
    eidx = lax.broadcasted_iota(jnp.int32, p.shape, 1)
    i1 = jnp.argmax(p, axis=-1)
    m1 = eidx == i1[:, None]
    p2 = jnp.where(m1, -jnp.inf, p)
    i2 = jnp.argmax(p2, axis=-1)
    m2 = eidx == i2[:, None]
    g1 = jnp.max(p, axis=-1, keepdims=True)
    g2 = jnp.max(p2, axis=-1, keepdims=True)
    gatev_ref[...] = jnp.concatenate([g1, g2], axis=1)

    m1f = m1.astype(jnp.float32)
    m2f = m2.astype(jnp.float32)
    o = jnp.concatenate([m1f[:, None, :], m2f[:, None, :]], axis=1)
    o = o.reshape(_NP, _E)
    o_scr[...] = o

    counts = jnp.sum(o, axis=0, keepdims=True)
    counts_ref[...] = counts.astype(jnp.int32)
    bpe = jnp.ceil(counts * (1.0 / _FB))
    tri_e = (lax.broadcasted_iota(jnp.int32, (_E, _E), 0)
             < lax.broadcasted_iota(jnp.int32, (_E, _E), 1)).astype(jnp.float32)
    start = float(_FB) * lax.dot_general(
        bpe, tri_e, (((1,), (0,)), ((), ())),
        preferred_element_type=jnp.float32)

    tri = (lax.broadcasted_iota(jnp.int32, (_TBLK, _TBLK), 1)
           < lax.broadcasted_iota(jnp.int32, (_TBLK, _TBLK), 0)).astype(jnp.float32)

    def body(c, carry):
        oc = o_scr[pl.ds(c * _TBLK, _TBLK), :]
        rc = carry + lax.dot_general(tri, oc, (((1,), (0,)), ((), ())),
                                     preferred_element_type=jnp.float32)
        dstc = jnp.sum(oc * (rc + start), axis=1, keepdims=True)
        dst_ref[pl.ds(c * _TBLK, _TBLK), :] = dstc.astype(jnp.int32)
        return carry + jnp.sum(oc, axis=0, keepdims=True)

    lax.fori_loop(0, _NP // _TBLK, body, jnp.zeros((1, _E), jnp.float32))


def _route(xf, W_router):
    return pl.pallas_call(
        _route_kernel,
        grid=(1,),
        in_specs=[
            pl.BlockSpec((_N, _D), lambda i: (0, 0)),
            pl.BlockSpec((_E, _D), lambda i: (0, 0)),
        ],
        out_specs=[
            pl.BlockSpec((_NP, 1), lambda i: (0, 0)),
            pl.BlockSpec((_N, _K), lambda i: (0, 0)),
            pl.BlockSpec((1, _E), lambda i: (0, 0)),
        ],
        out_shape=[
            jax.ShapeDtypeStruct((_NP, 1), jnp.int32),
            jax.ShapeDtypeStruct((_N, _K), jnp.float32),
            jax.ShapeDtypeStruct((1, _E), jnp.int32),
        ],
        scratch_shapes=[pltpu.VMEM((_NP, _E), jnp.float32)],
    )(xf, W_router)


_SC_CH = 128
_PW = 128
_R = _D // _PW


def _sc_dispatch(x6, dst6, tok6):
    info = plsc.get_sparse_core_info()
    nw = info.num_cores * info.num_subcores
    pairs_per_w = _NP // nw
    nch = pairs_per_w // _SC_CH
    cp = _SC_CH * _R
    mesh = plsc.VectorSubcoreMesh(core_axis_name="c", subcore_axis_name="s")

    @functools.partial(
        pl.kernel, mesh=mesh,
        out_type=jax.ShapeDtypeStruct((_NS * _R, _PW), jnp.float32),
        scratch_types=[
            pltpu.VMEM((cp,), jnp.int32),
            pltpu.VMEM((cp,), jnp.int32),
            pltpu.VMEM((cp, _PW), jnp.float32),
            pltpu.SemaphoreType.DMA,
        ],
    )
    def k(x_hbm, dst_hbm, tok_hbm, xs_hbm, tok_v, dst_v, rows_v, sem):
        wid = lax.axis_index("s") * info.num_cores + lax.axis_index("c")
        base = wid * pairs_per_w * _R

        for c in range(nch):
            cbase = base + c * cp
            pltpu.sync_copy(tok_hbm.at[pl.ds(cbase, cp)], tok_v)
            pltpu.sync_copy(dst_hbm.at[pl.ds(cbase, cp)], dst_v)
            pltpu.async_copy(x_hbm.at[tok_v], rows_v, sem).wait()
            pltpu.async_copy(rows_v, xs_hbm.at[dst_v], sem).wait()

    return k(x6, dst6, tok6)


def _ffn_kernel(be_ref, xs_ref, w1_ref, b1_ref, w2_ref, b2_ref, po_ref):
    xb = xs_ref[...]
    h = lax.dot_general(xb, w1_ref[0], (((1,), (1,)), ((), ())),
                        preferred_element_type=jnp.float32) + b1_ref[0]
    h = 0.5 * h * (1.0 + lax.erf(h * 0.7071067811865476))
    po_ref[...] = lax.dot_general(h, w2_ref[0], (((1,), (1,)), ((), ())),
                                  preferred_element_type=jnp.float32) + b2_ref[0]


def _ffn(block_expert, xs, W1, b1, W2, b2):
    grid_spec = pltpu.PrefetchScalarGridSpec(
        num_scalar_prefetch=1,
        grid=(_NB,),
        in_specs=[
            pl.BlockSpec((_FB, _D), lambda b, be: (b, 0)),
            pl.BlockSpec((1, _F, _D), lambda b, be: (be[b], 0, 0)),
            pl.BlockSpec((1, 1, _F), lambda b, be: (be[b], 0, 0)),
            pl.BlockSpec((1, _D, _F), lambda b, be: (be[b], 0, 0)),
            pl.BlockSpec((1, 1, _D), lambda b, be: (be[b], 0, 0)),
        ],
        out_specs=pl.BlockSpec((_FB, _D), lambda b, be: (b, 0)),
    )
    return pl.pallas_call(
        _ffn_kernel,
        grid_spec=grid_spec,
        out_shape=jax.ShapeDtypeStruct((_NS, _D), jnp.float32),
    )(block_expert, xs, W1, b1.reshape(_E, 1, _F), W2, b2.reshape(_E, 1, _D))


_CB_T = 32


def _sc_combine(x6, po6, dst6, gates16):
    info = plsc.get_sparse_core_info()
    nw = info.num_cores * info.num_subcores
    tok_per_w = _N // nw
    nch = tok_per_w // _CB_T
    tp = _CB_T * _R
    pp = 2 * _CB_T * _R
    mesh = plsc.VectorSubcoreMesh(core_axis_name="c", subcore_axis_name="s")

    @functools.partial(
        pl.kernel, mesh=mesh,
        out_type=jax.ShapeDtypeStruct((_N * _R, _PW), jnp.float32),
        scratch_types=[
            pltpu.VMEM((pp,), jnp.int32),
            pltpu.VMEM((2 * _CB_T, 16), jnp.float32),
            pltpu.VMEM((pp, _PW), jnp.float32),
            pltpu.VMEM((tp, _PW), jnp.float32),
            pltpu.SemaphoreType.DMA,
        ],
    )
    def k(x_hbm, po_hbm, dst_hbm, g_hbm, y_hbm, dst_v, g_v, po_v, acc_v, sem):
        wid = lax.axis_index("s") * info.num_cores + lax.axis_index("c")
        base_t = wid * tok_per_w

        def chunk(c, _):
            tb = base_t + c * _CB_T
            pltpu.sync_copy(dst_hbm.at[pl.ds(2 * tb * _R, pp)], dst_v)
            pltpu.sync_copy(g_hbm.at[pl.ds(2 * tb, 2 * _CB_T), :], g_v)
            pltpu.async_copy(po_hbm.at[dst_v], po_v, sem).wait()
            pltpu.sync_copy(x_hbm.at[pl.ds(tb * _R, tp)], acc_v)

            def tok(j, _2):
                ga = g_v[2 * j, :]
                gb = g_v[2 * j + 1, :]
                for r in range(_R):
                    for v in range(8):
                        sl = pl.ds(v * 16, 16)
                        acc_v[j * _R + r, sl] = (
                            acc_v[j * _R + r, sl]
                            + ga * po_v[2 * j * _R + r, sl]
                            + gb * po_v[(2 * j + 1) * _R + r, sl])
                return 0

            lax.fori_loop(0, _CB_T, tok, 0)
            pltpu.sync_copy(acc_v, y_hbm.at[pl.ds(tb * _R, tp)])
            return 0

        lax.fori_loop(0, nch, chunk, 0)

    return k(x6, po6, dst6, gates16)


def kernel(x, W_router, W1, b1, W2, b2):
    B, L, D = x.shape
    xf = x.reshape(_N, D)

    dst2d, gatev, counts = _route(xf, W_router)
    dst = dst2d.reshape(_NP)
    gates = gatev.reshape(_NP)

    cnt = counts.reshape(_E)
    cb = jnp.cumsum((cnt + _FB - 1) // _FB)
    block_expert = jnp.clip(
        jnp.searchsorted(cb, jnp.arange(_NB, dtype=jnp.int32), side="right"),
        0, _E - 1).astype(jnp.int32)

    piece = jnp.arange(_R, dtype=jnp.int32)
    tok6 = ((jnp.arange(_NP, dtype=jnp.int32) // _K * _R)[:, None]
            + piece).reshape(-1)
    dst6 = ((dst * _R)[:, None] + piece).reshape(-1)
    gates16 = jnp.broadcast_to(gates[:, None], (_NP, 16))

    x6 = xf.reshape(_N * _R, _PW)
    xs = _sc_dispatch(x6, dst6, tok6)
    po = _ffn(block_expert, xs.reshape(_NS, _D), W1, b1, W2, b2)
    y = _sc_combine(x6, po.reshape(_NS * _R, _PW), dst6, gates16)
    return y.reshape(B, L, D)


# speedup vs baseline: 1.3340x; 1.0376x over previous
import functools

import jax
import jax.numpy as jnp
from jax import lax
from jax.experimental import pallas as pl
from jax.experimental.pallas import tpu as pltpu
from jax.experimental.pallas import tpu_sc as plsc

_E = 16
_D = 768
_F = 4 * _D
_N = 4096
_K = 2
_NP = _N * _K
_TBLK = 256
_FB = 256
_NB = _NP // _FB + _E
_NS = _NB * _FB


def _route_kernel(x_ref, wr_ref, dst_ref, gatev_ref, counts_ref, o_scr):
    x = x_ref[...]
    logits = lax.dot_general(x, wr_ref[...], (((1,), (1,)), ((), ())),
                             preferred_element_type=jnp.float32)
    m = jnp.max(logits, axis=-1, keepdims=True)
    p = jnp.exp(logits - m)
    p = p / jnp.sum(p, axis=-1, keepdims=True)
    eidx = lax.broadcasted_iota(jnp.int32, p.shape, 1)
    i1 = jnp.argmax(p, axis=-1)
    m1 = eidx == i1[:, None]
    p2 = jnp.where(m1, -jnp.inf, p)
    i2 = jnp.argmax(p2, axis=-1)
    m2 = eidx == i2[:, None]
    g1 = jnp.max(p, axis=-1, keepdims=True)
    g2 = jnp.max(p2, axis=-1, keepdims=True)
    gatev_ref[...] = jnp.concatenate([g1, g2], axis=1)

    m1f = m1.astype(jnp.float32)
    m2f = m2.astype(jnp.float32)
    o = jnp.concatenate([m1f[:, None, :], m2f[:, None, :]], axis=1)
    o = o.reshape(_NP, _E)
    o_scr[...] = o

    counts = jnp.sum(o, axis=0, keepdims=True)
    counts_ref[...] = counts.astype(jnp.int32)
    bpe = jnp.ceil(counts * (1.0 / _FB))
    tri_e = (lax.broadcasted_iota(jnp.int32, (_E, _E), 0)
             < lax.broadcasted_iota(jnp.int32, (_E, _E), 1)).astype(jnp.float32)
    start = float(_FB) * lax.dot_general(
        bpe, tri_e, (((1,), (0,)), ((), ())),
        preferred_element_type=jnp.float32)

    tri = (lax.broadcasted_iota(jnp.int32, (_TBLK, _TBLK), 1)
           < lax.broadcasted_iota(jnp.int32, (_TBLK, _TBLK), 0)).astype(jnp.float32)

    def body(c, carry):
        oc = o_scr[pl.ds(c * _TBLK, _TBLK), :]
        rc = carry + lax.dot_general(tri, oc, (((1,), (0,)), ((), ())),
                                     preferred_element_type=jnp.float32)
        dstc = jnp.sum(oc * (rc + start), axis=1, keepdims=True)
        dst_ref[pl.ds(c * _TBLK, _TBLK), :] = dstc.astype(jnp.int32)
        return carry + jnp.sum(oc, axis=0, keepdims=True)

    lax.fori_loop(0, _NP // _TBLK, body, jnp.zeros((1, _E), jnp.float32))


def _route(xf, W_router):
    return pl.pallas_call(
        _route_kernel,
        grid=(1,),
        in_specs=[
            pl.BlockSpec((_N, _D), lambda i: (0, 0)),
            pl.BlockSpec((_E, _D), lambda i: (0, 0)),
        ],
        out_specs=[
            pl.BlockSpec((_NP, 1), lambda i: (0, 0)),
            pl.BlockSpec((_N, _K), lambda i: (0, 0)),
            pl.BlockSpec((1, _E), lambda i: (0, 0)),
        ],
        out_shape=[
            jax.ShapeDtypeStruct((_NP, 1), jnp.int32),
            jax.ShapeDtypeStruct((_N, _K), jnp.float32),
            jax.ShapeDtypeStruct((1, _E), jnp.int32),
        ],
        scratch_shapes=[pltpu.VMEM((_NP, _E), jnp.float32)],
    )(xf, W_router)


_SC_CH = 128
_PW = 128
_R = _D // _PW


def _sc_dispatch(x6, dst6, tok6):
    info = plsc.get_sparse_core_info()
    nw = info.num_cores * info.num_subcores
    pairs_per_w = _NP // nw
    nch = pairs_per_w // _SC_CH
    cp = _SC_CH * _R
    mesh = plsc.VectorSubcoreMesh(core_axis_name="c", subcore_axis_name="s")

    @functools.partial(
        pl.kernel, mesh=mesh,
        out_type=jax.ShapeDtypeStruct((_NS * _R, _PW), jnp.float32),
        scratch_types=[
            pltpu.VMEM((cp,), jnp.int32),
            pltpu.VMEM((cp,), jnp.int32),
            pltpu.VMEM((cp, _PW), jnp.float32),
            pltpu.SemaphoreType.DMA,
        ],
    )
    def k(x_hbm, dst_hbm, tok_hbm, xs_hbm, tok_v, dst_v, rows_v, sem):
        wid = lax.axis_index("s") * info.num_cores + lax.axis_index("c")
        base = wid * pairs_per_w * _R

        for c in range(nch):
            cbase = base + c * cp
            pltpu.sync_copy(tok_hbm.at[pl.ds(cbase, cp)], tok_v)
            pltpu.sync_copy(dst_hbm.at[pl.ds(cbase, cp)], dst_v)
            pltpu.async_copy(x_hbm.at[tok_v], rows_v, sem).wait()
            pltpu.async_copy(rows_v, xs_hbm.at[dst_v], sem).wait()

    return k(x6, dst6, tok6)


def _ffn_kernel(be_ref, xs_ref, w1_ref, b1_ref, w2_ref, b2_ref, po_ref):
    xb = xs_ref[...]
    h = lax.dot_general(xb, w1_ref[0], (((1,), (1,)), ((), ())),
                        preferred_element_type=jnp.float32) + b1_ref[0]
    h = 0.5 * h * (1.0 + lax.erf(h * 0.7071067811865476))
    po_ref[...] = lax.dot_general(h, w2_ref[0], (((1,), (1,)), ((), ())),
                                  preferred_element_type=jnp.float32) + b2_ref[0]


def _ffn(block_expert, xs, W1, b1, W2, b2):
    grid_spec = pltpu.PrefetchScalarGridSpec(
        num_scalar_prefetch=1,
        grid=(_NB,),
        in_specs=[
            pl.BlockSpec((_FB, _D), lambda b, be: (b, 0)),
            pl.BlockSpec((1, _F, _D), lambda b, be: (be[b], 0, 0)),
            pl.BlockSpec((1, 1, _F), lambda b, be: (be[b], 0, 0)),
            pl.BlockSpec((1, _D, _F), lambda b, be: (be[b], 0, 0)),
            pl.BlockSpec((1, 1, _D), lambda b, be: (be[b], 0, 0)),
        ],
        out_specs=pl.BlockSpec((_FB, _D), lambda b, be: (b, 0)),
    )
    return pl.pallas_call(
        _ffn_kernel,
        grid_spec=grid_spec,
        out_shape=jax.ShapeDtypeStruct((_NS, _D), jnp.float32),
    )(block_expert, xs, W1, b1.reshape(_E, 1, _F), W2, b2.reshape(_E, 1, _D))


_CB_T = 32


def _sc_combine(x6, po6, dst6, gates16):
    info = plsc.get_sparse_core_info()
    nw = info.num_cores * info.num_subcores
    tok_per_w = _N // nw
    nch = tok_per_w // _CB_T
    tp = _CB_T * _R
    pp = 2 * _CB_T * _R
    mesh = plsc.VectorSubcoreMesh(core_axis_name="c", subcore_axis_name="s")

    @functools.partial(
        pl.kernel, mesh=mesh,
        out_type=jax.ShapeDtypeStruct((_N * _R, _PW), jnp.float32),
        scratch_types=[
            pltpu.VMEM((pp,), jnp.int32),
            pltpu.VMEM((2 * _CB_T, 16), jnp.float32),
            pltpu.VMEM((pp, _PW), jnp.float32),
            pltpu.VMEM((tp, _PW), jnp.float32),
            pltpu.SemaphoreType.DMA,
        ],
    )
    def k(x_hbm, po_hbm, dst_hbm, g_hbm, y_hbm, dst_v, g_v, po_v, acc_v, sem):
        wid = lax.axis_index("s") * info.num_cores + lax.axis_index("c")
        base_t = wid * tok_per_w

        def chunk(c, _):
            tb = base_t + c * _CB_T
            pltpu.sync_copy(dst_hbm.at[pl.ds(2 * tb * _R, pp)], dst_v)
            pltpu.sync_copy(g_hbm.at[pl.ds(2 * tb, 2 * _CB_T), :], g_v)
            pltpu.async_copy(po_hbm.at[dst_v], po_v, sem).wait()
            pltpu.sync_copy(x_hbm.at[pl.ds(tb * _R, tp)], acc_v)

            def tok(j, _2):
                ga = g_v[2 * j, :]
                gb = g_v[2 * j + 1, :]
                for r in range(_R):
                    for v in range(8):
                        sl = pl.ds(v * 16, 16)
                        acc_v[j * _R + r, sl] = (
                            acc_v[j * _R + r, sl]
                            + ga * po_v[2 * j * _R + r, sl]
                            + gb * po_v[(2 * j + 1) * _R + r, sl])
                return 0

            lax.fori_loop(0, _CB_T, tok, 0)
            pltpu.sync_copy(acc_v, y_hbm.at[pl.ds(tb * _R, tp)])
            return 0

        lax.fori_loop(0, nch, chunk, 0)

    return k(x6, po6, dst6, gates16)


def kernel(x, W_router, W1, b1, W2, b2):
    B, L, D = x.shape
    xf = x.reshape(_N, D)

    dst2d, gatev, counts = _route(xf, W_router)
    dst = dst2d.reshape(_NP)
    gates = gatev.reshape(_NP)

    cnt = counts.reshape(_E)
    cb = jnp.cumsum((cnt + _FB - 1) // _FB)
    block_expert = jnp.clip(
        jnp.searchsorted(cb, jnp.arange(_NB, dtype=jnp.int32), side="right"),
        0, _E - 1).astype(jnp.int32)

    piece = jnp.arange(_R, dtype=jnp.int32)
    tok6 = ((jnp.arange(_NP, dtype=jnp.int32) // _K * _R)[:, None]
            + piece).reshape(-1)
    dst6 = ((dst * _R)[:, None] + piece).reshape(-1)
    gates16 = jnp.broadcast_to(gates[:, None], (_NP, 16))

    x6 = xf.reshape(_N * _R, _PW)
    xs = _sc_dispatch(x6, dst6, tok6)
    po = _ffn(block_expert, xs.reshape(_NS, _D), W1, b1, W2, b2)
    y = _sc_combine(x6, po.reshape(_NS * _R, _PW), dst6, gates16)
    return y.reshape(B, L, D)
